# Initial kernel scaffold; baseline (speedup 1.0000x reference)
import jax
import jax.numpy as jnp
from jax.experimental import pallas as pl


def kernel(token_ids, embedding):
    raise NotImplementedError("write your pallas kernel here")


# speedup vs baseline: 1.5072x; 1.5072x over previous
import functools

import jax
import jax.numpy as jnp
from jax import lax
from jax.experimental import pallas as pl
from jax.experimental.pallas import tpu as pltpu
from jax.experimental.pallas import tpu_sc as plsc

D_MODEL = 768
N_TOKENS = 4 * 8192

_NUM_CORES = 2
_NUM_SUBCORES = 16
_NW = _NUM_CORES * _NUM_SUBCORES
_B_PER_W = N_TOKENS // _NW
_CHUNK = 64
_N_CHUNKS = _B_PER_W // _CHUNK


def _emb_body(table_hbm, idx_hbm, out_hbm, idx_v, rows_v, sem):
    wid = lax.axis_index("s") * _NUM_CORES + lax.axis_index("c")
    base = wid * _B_PER_W
    pltpu.sync_copy(idx_hbm.at[pl.ds(base, _B_PER_W)], idx_v)
    for c in range(_N_CHUNKS):
        cbase = c * _CHUNK
        pltpu.async_copy(
            table_hbm.at[idx_v.at[pl.ds(cbase, _CHUNK)]], rows_v, sem
        ).wait()
        pltpu.sync_copy(rows_v, out_hbm.at[pl.ds(base + cbase, _CHUNK)])


@jax.jit
def _embed_flat(idx_flat, embedding):
    mesh = plsc.VectorSubcoreMesh(core_axis_name="c", subcore_axis_name="s")
    k = functools.partial(
        pl.kernel,
        mesh=mesh,
        out_type=jax.ShapeDtypeStruct((N_TOKENS, D_MODEL), jnp.float32),
        scratch_types=[
            pltpu.VMEM((_B_PER_W,), jnp.int32),
            pltpu.VMEM((_CHUNK, D_MODEL), jnp.float32),
            pltpu.SemaphoreType.DMA,
        ],
    )(_emb_body)
    return k(embedding, idx_flat)


def kernel(token_ids, embedding):
    idx_flat = token_ids.reshape(-1).astype(jnp.int32)
    out = _embed_flat(idx_flat, embedding)
    return out.reshape(token_ids.shape[0], token_ids.shape[1], D_MODEL)


# speedup vs baseline: 1.6622x; 1.1029x over previous
import functools

import jax
import jax.numpy as jnp
from jax import lax
from jax.experimental import pallas as pl
from jax.experimental.pallas import tpu as pltpu
from jax.experimental.pallas import tpu_sc as plsc

D_MODEL = 768
N_TOKENS = 4 * 8192

_NUM_CORES = 2
_NUM_SUBCORES = 16
_NW = _NUM_CORES * _NUM_SUBCORES
_B_PER_W = N_TOKENS // _NW
_CHUNK = 64
_N_CHUNKS = _B_PER_W // _CHUNK


def _emb_body(table_hbm, idx_hbm, out_hbm, idx_v, rows0, rows1, g0, g1, s0, s1):
    wid = lax.axis_index("s") * _NUM_CORES + lax.axis_index("c")
    base = wid * _B_PER_W
    rows = (rows0, rows1)
    gsem = (g0, g1)
    ssem = (s0, s1)
    pltpu.sync_copy(idx_hbm.at[pl.ds(base, _B_PER_W)], idx_v)

    def gather(c, b):
        return pltpu.async_copy(
            table_hbm.at[idx_v.at[pl.ds(c * _CHUNK, _CHUNK)]], rows[b], gsem[b]
        )

    def store(c, b):
        return pltpu.async_copy(
            rows[b], out_hbm.at[pl.ds(base + c * _CHUNK, _CHUNK)], ssem[b]
        )

    pending = [gather(0, 0), gather(1, 1)]
    stores = [None, None]
    for c in range(_N_CHUNKS):
        b = c % 2
        pending[b].wait()
        st = store(c, b)
        if c + 2 < _N_CHUNKS:
            st.wait()
            pending[b] = gather(c + 2, b)
        else:
            stores[b] = st
    stores[0].wait()
    stores[1].wait()


@jax.jit
def _embed_flat(idx_flat, embedding):
    mesh = plsc.VectorSubcoreMesh(core_axis_name="c", subcore_axis_name="s")
    k = functools.partial(
        pl.kernel,
        mesh=mesh,
        out_type=jax.ShapeDtypeStruct((N_TOKENS, D_MODEL), jnp.float32),
        scratch_types=[
            pltpu.VMEM((_B_PER_W,), jnp.int32),
            pltpu.VMEM((_CHUNK, D_MODEL), jnp.float32),
            pltpu.VMEM((_CHUNK, D_MODEL), jnp.float32),
            pltpu.SemaphoreType.DMA,
            pltpu.SemaphoreType.DMA,
            pltpu.SemaphoreType.DMA,
            pltpu.SemaphoreType.DMA,
        ],
    )(_emb_body)
    return k(embedding, idx_flat)


def kernel(token_ids, embedding):
    idx_flat = token_ids.reshape(-1).astype(jnp.int32)
    out = _embed_flat(idx_flat, embedding)
    return out.reshape(token_ids.shape[0], token_ids.shape[1], D_MODEL)


# speedup vs baseline: 1.7145x; 1.0314x over previous
import functools

import jax
import jax.numpy as jnp
from jax import lax
from jax.experimental import pallas as pl
from jax.experimental.pallas import tpu as pltpu
from jax.experimental.pallas import tpu_sc as plsc

D_MODEL = 768
N_TOKENS = 4 * 8192

_NUM_CORES = 2
_NUM_SUBCORES = 16
_NW = _NUM_CORES * _NUM_SUBCORES
_B_PER_W = N_TOKENS // _NW
_CHUNK = 32
_N_CHUNKS = _B_PER_W // _CHUNK
_ND = 4
_N_GROUPS = _N_CHUNKS // _ND


def _emb_body(table_hbm, idx_hbm, out_hbm, idx_v, rows0, rows1, rows2, rows3,
              g0, g1, g2, g3, s0, s1, s2, s3):
    wid = lax.axis_index("s") * _NUM_CORES + lax.axis_index("c")
    base = wid * _B_PER_W
    rows = (rows0, rows1, rows2, rows3)
    gsem = (g0, g1, g2, g3)
    ssem = (s0, s1, s2, s3)
    pltpu.sync_copy(idx_hbm.at[pl.ds(base, _B_PER_W)], idx_v)

    def gather_start(c, b):
        pltpu.async_copy(
            table_hbm.at[idx_v.at[pl.ds(c * _CHUNK, _CHUNK)]], rows[b], gsem[b]
        )

    def gather_wait(b):
        pltpu.make_async_copy(
            table_hbm.at[idx_v.at[pl.ds(0, _CHUNK)]], rows[b], gsem[b]
        ).wait()

    def store_start(c, b):
        pltpu.async_copy(
            rows[b], out_hbm.at[pl.ds(base + c * _CHUNK, _CHUNK)], ssem[b]
        )

    def store_wait(b):
        pltpu.make_async_copy(
            rows[b], out_hbm.at[pl.ds(base, _CHUNK)], ssem[b]
        ).wait()

    for b in range(_ND):
        gather_start(b, b)

    def group(g, _):
        c0 = g * _ND
        for b in range(_ND):
            gather_wait(b)
            store_start(c0 + b, b)
            store_wait(b)
            gather_start(c0 + b + _ND, b)
        return 0

    lax.fori_loop(0, _N_GROUPS - 1, group, 0, unroll=False)
    c0 = (_N_GROUPS - 1) * _ND
    for b in range(_ND):
        gather_wait(b)
        store_start(c0 + b, b)
    for b in range(_ND):
        store_wait(b)


@jax.jit
def _embed_flat(idx_flat, embedding):
    mesh = plsc.VectorSubcoreMesh(core_axis_name="c", subcore_axis_name="s")
    k = functools.partial(
        pl.kernel,
        mesh=mesh,
        out_type=jax.ShapeDtypeStruct((N_TOKENS, D_MODEL), jnp.float32),
        scratch_types=(
            [pltpu.VMEM((_B_PER_W,), jnp.int32)]
            + [pltpu.VMEM((_CHUNK, D_MODEL), jnp.float32)] * _ND
            + [pltpu.SemaphoreType.DMA] * (2 * _ND)
        ),
    )(_emb_body)
    return k(embedding, idx_flat)


def kernel(token_ids, embedding):
    idx_flat = token_ids.reshape(-1).astype(jnp.int32)
    out = _embed_flat(idx_flat, embedding)
    return out.reshape(token_ids.shape[0], token_ids.shape[1], D_MODEL)
